# Initial kernel scaffold; baseline (speedup 1.0000x reference)
import jax
import jax.numpy as jnp
from jax.experimental import pallas as pl


def kernel(x, edge_index, edge_attr, lin0_W, lin0_b, bn0_g, bn0_b, nn1_W1, nn1_b1, nn1_bn_g, nn1_bn_b, nn1_W2, nn1_b2, bn1_g, bn1_b, lin1_W, lin1_b):
    raise NotImplementedError("write your pallas kernel here")


# speedup vs baseline: 1.2234x; 1.2234x over previous
import functools

import jax
import jax.numpy as jnp
from jax import lax
from jax.experimental import pallas as pl
from jax.experimental.pallas import tpu as pltpu
from jax.experimental.pallas import tpu_sc as plsc

N_NODES = 10000
N_EDGES = 160000
IN_DIM = 128
NUM_TYPES = 16
DIM = 32
EPS = 1e-5

NC, NS = 2, 16
NW = NC * NS
EW = N_EDGES // NW
SUB = 125
NSUB = EW // SUB
CHUNK = 1000
NCHUNK = EW // CHUNK
SPC = NSUB // NCHUNK
N_PAD = 10240
ROWS_PER_SUBCORE = N_PAD // NS

_MSG_T = 1000


def _lin0_body(x_ref, w_ref, b_ref, g_ref, bb_ref, o_ref):
    hp = jnp.dot(x_ref[...], w_ref[...], preferred_element_type=jnp.float32)
    hp = hp + b_ref[...]
    mu = jnp.mean(hp, axis=0, keepdims=True)
    d = hp - mu
    var = jnp.mean(d * d, axis=0, keepdims=True)
    o_ref[...] = jnp.maximum(d * lax.rsqrt(var + EPS) * g_ref[...] + bb_ref[...], 0.0)


def _epre_body(ea_ref, w_ref, b_ref, o_ref, s1_ref, s2_ref):
    ep = jnp.dot(ea_ref[...], w_ref[...], preferred_element_type=jnp.float32)
    ep = ep + b_ref[...]
    o_ref[...] = ep

    @pl.when(pl.program_id(0) == 0)
    def _():
        s1_ref[...] = jnp.zeros_like(s1_ref)
        s2_ref[...] = jnp.zeros_like(s2_ref)

    s1_ref[...] += jnp.sum(ep, axis=0, keepdims=True)
    s2_ref[...] += jnp.sum(ep * ep, axis=0, keepdims=True)


def _gather_body(h_hbm, src_hbm, xj_hbm, idx_v, rows_v, sem):
    c = lax.axis_index("c")
    s = lax.axis_index("s")
    wid = c * NS + s
    pltpu.sync_copy(src_hbm.at[wid], idx_v)

    def chunk(ci, carry):
        cps = [
            pltpu.async_copy(
                h_hbm.at[idx_v.at[ci * SPC + j]],
                rows_v.at[pl.ds(j * SUB, SUB)],
                sem,
            )
            for j in range(SPC)
        ]
        for cp in cps:
            cp.wait()
        pltpu.sync_copy(rows_v, xj_hbm.at[pl.ds(wid * EW + ci * CHUNK, CHUNK)])
        return carry

    lax.fori_loop(0, NCHUNK, chunk, 0)


def _msg_body(xj_ref, ep_ref, s1_ref, s2_ref, g_ref, b_ref, w2b_ref, b2_ref,
              o_ref):
    inv_e = 1.0 / N_EDGES
    mu = s1_ref[...] * inv_e
    var = s2_ref[...] * inv_e - mu * mu
    ef = jnp.maximum((ep_ref[...] - mu) * lax.rsqrt(var + EPS) * g_ref[...]
                     + b_ref[...], 0.0)
    xj = xj_ref[...]
    ext = jnp.concatenate([xj[:, d:d + 1] * ef for d in range(DIM)], axis=1)
    o_ref[...] = (
        jnp.dot(ext, w2b_ref[...], preferred_element_type=jnp.float32)
        + jnp.dot(xj, b2_ref[...], preferred_element_type=jnp.float32)
    )


def _scatter_body(msg_hbm, dst_hbm, zeros_hbm, out_hbm, idx_v, buf, zbuf, acc,
                  sem):
    c = lax.axis_index("c")
    s = lax.axis_index("s")
    wid = c * NS + s
    pltpu.sync_copy(zeros_hbm, zbuf)
    pltpu.sync_copy(zbuf, acc.at[pl.ds(s * ROWS_PER_SUBCORE, ROWS_PER_SUBCORE)])
    plsc.subcore_barrier()

    pltpu.sync_copy(dst_hbm.at[wid], idx_v)

    def chunk(ci, carry):
        pltpu.sync_copy(msg_hbm.at[pl.ds(wid * EW + ci * CHUNK, CHUNK)], buf)

        def sub(j, carry2):
            pltpu.sync_copy(
                buf.at[pl.ds(j * SUB, SUB)],
                acc.at[idx_v.at[ci * SPC + j]],
                add=True,
            )
            return carry2

        lax.fori_loop(0, SPC, sub, 0)
        return carry

    lax.fori_loop(0, NCHUNK, chunk, 0)
    plsc.subcore_barrier()
    pltpu.sync_copy(
        acc.at[pl.ds(s * ROWS_PER_SUBCORE, ROWS_PER_SUBCORE)],
        out_hbm.at[c, pl.ds(s * ROWS_PER_SUBCORE, ROWS_PER_SUBCORE)],
    )


def _final_body(p_ref, g_ref, bb_ref, w_ref, b_ref, o_ref):
    agg = p_ref[0, :N_NODES, :] + p_ref[1, :N_NODES, :]
    mu = jnp.mean(agg, axis=0, keepdims=True)
    d = agg - mu
    var = jnp.mean(d * d, axis=0, keepdims=True)
    h2 = jnp.maximum(d * lax.rsqrt(var + EPS) * g_ref[...] + bb_ref[...], 0.0)
    s = jnp.sum(h2 * w_ref[...])
    o_ref[...] = jnp.full((1, 1), 1.0 / N_NODES) * s + b_ref[...]


def kernel(x, edge_index, edge_attr,
           lin0_W, lin0_b, bn0_g, bn0_b,
           nn1_W1, nn1_b1, nn1_bn_g, nn1_bn_b, nn1_W2, nn1_b2,
           bn1_g, bn1_b, lin1_W, lin1_b):
    f32 = jnp.float32
    src3 = edge_index[0].astype(jnp.int32).reshape(NW, NSUB, SUB)
    dst3 = edge_index[1].astype(jnp.int32).reshape(NW, NSUB, SUB)
    w2b = nn1_W2.reshape(DIM, DIM, DIM).transpose(1, 0, 2).reshape(DIM * DIM, DIM)
    b2m = nn1_b2.reshape(DIM, DIM)

    h = pl.pallas_call(
        _lin0_body,
        out_shape=jax.ShapeDtypeStruct((N_NODES, DIM), f32),
    )(x, lin0_W, lin0_b.reshape(1, DIM), bn0_g.reshape(1, DIM),
      bn0_b.reshape(1, DIM))

    t2 = 2000
    g2 = N_EDGES // t2
    epre, s1, s2 = pl.pallas_call(
        _epre_body,
        grid=(g2,),
        in_specs=[
            pl.BlockSpec((t2, NUM_TYPES), lambda i: (i, 0)),
            pl.BlockSpec((NUM_TYPES, DIM), lambda i: (0, 0)),
            pl.BlockSpec((1, DIM), lambda i: (0, 0)),
        ],
        out_specs=[
            pl.BlockSpec((t2, DIM), lambda i: (i, 0)),
            pl.BlockSpec((1, DIM), lambda i: (0, 0)),
            pl.BlockSpec((1, DIM), lambda i: (0, 0)),
        ],
        out_shape=[
            jax.ShapeDtypeStruct((N_EDGES, DIM), f32),
            jax.ShapeDtypeStruct((1, DIM), f32),
            jax.ShapeDtypeStruct((1, DIM), f32),
        ],
    )(edge_attr, nn1_W1, nn1_b1.reshape(1, DIM))

    mesh = plsc.VectorSubcoreMesh(core_axis_name="c", subcore_axis_name="s",
                                  num_cores=NC, num_subcores=NS)
    sc_params = pltpu.CompilerParams(use_tc_tiling_on_sc=False)
    xj = pl.kernel(
        _gather_body,
        out_type=jax.ShapeDtypeStruct((N_EDGES, DIM), f32),
        mesh=mesh,
        compiler_params=sc_params,
        scratch_types=[
            pltpu.VMEM((NSUB, SUB), jnp.int32),
            pltpu.VMEM((CHUNK, DIM), f32),
            pltpu.SemaphoreType.DMA,
        ],
    )(h, src3)

    gm = N_EDGES // _MSG_T
    msg = pl.pallas_call(
        _msg_body,
        grid=(gm,),
        in_specs=[
            pl.BlockSpec((_MSG_T, DIM), lambda i: (i, 0)),
            pl.BlockSpec((_MSG_T, DIM), lambda i: (i, 0)),
            pl.BlockSpec((1, DIM), lambda i: (0, 0)),
            pl.BlockSpec((1, DIM), lambda i: (0, 0)),
            pl.BlockSpec((1, DIM), lambda i: (0, 0)),
            pl.BlockSpec((1, DIM), lambda i: (0, 0)),
            pl.BlockSpec((DIM * DIM, DIM), lambda i: (0, 0)),
            pl.BlockSpec((DIM, DIM), lambda i: (0, 0)),
        ],
        out_specs=pl.BlockSpec((_MSG_T, DIM), lambda i: (i, 0)),
        out_shape=jax.ShapeDtypeStruct((N_EDGES, DIM), f32),
    )(xj, epre, s1, s2, nn1_bn_g.reshape(1, DIM), nn1_bn_b.reshape(1, DIM),
      w2b, b2m)

    partials = pl.kernel(
        _scatter_body,
        out_type=jax.ShapeDtypeStruct((NC, N_PAD, DIM), f32),
        mesh=mesh,
        compiler_params=sc_params,
        scratch_types=[
            pltpu.VMEM((NSUB, SUB), jnp.int32),
            pltpu.VMEM((CHUNK, DIM), f32),
            pltpu.VMEM((ROWS_PER_SUBCORE, DIM), f32),
            pltpu.VMEM_SHARED((N_PAD, DIM), f32),
            pltpu.SemaphoreType.DMA,
        ],
    )(msg, dst3, jnp.zeros((ROWS_PER_SUBCORE, DIM), f32))

    out2d = pl.pallas_call(
        _final_body,
        out_shape=jax.ShapeDtypeStruct((1, 1), f32),
    )(partials, bn1_g.reshape(1, DIM), bn1_b.reshape(1, DIM),
      lin1_W.reshape(1, DIM), lin1_b.reshape(1, 1))
    return out2d[0, 0]


# speedup vs baseline: 2.9274x; 2.3928x over previous
import functools

import jax
import jax.numpy as jnp
from jax import lax
from jax.experimental import pallas as pl
from jax.experimental.pallas import tpu as pltpu
from jax.experimental.pallas import tpu_sc as plsc

N_NODES = 10000
N_EDGES = 160000
IN_DIM = 128
NUM_TYPES = 16
DIM = 32
EPS = 1e-5

NC, NS = 2, 16
NW = NC * NS
EW = N_EDGES // NW
SUB = 125
NSUB = EW // SUB
CHUNK = 1000
NCHUNK = EW // CHUNK
SPC = NSUB // NCHUNK
N_PAD = 10240
ROWS_PER_SUBCORE = N_PAD // NS

_MSG_T = 1000


def _lin0_body(x_ref, w_ref, b_ref, g_ref, bb_ref, o_ref):
    hp = jnp.dot(x_ref[...], w_ref[...], preferred_element_type=jnp.float32)
    hp = hp + b_ref[...]
    mu = jnp.mean(hp, axis=0, keepdims=True)
    d = hp - mu
    var = jnp.mean(d * d, axis=0, keepdims=True)
    o_ref[...] = jnp.maximum(d * lax.rsqrt(var + EPS) * g_ref[...] + bb_ref[...], 0.0)


def _epre_body(ea_ref, w_ref, b_ref, o_ref, s1_ref, s2_ref):
    ep = jnp.dot(ea_ref[...], w_ref[...], preferred_element_type=jnp.float32)
    ep = ep + b_ref[...]
    o_ref[...] = ep

    @pl.when(pl.program_id(0) == 0)
    def _():
        s1_ref[...] = jnp.zeros_like(s1_ref)
        s2_ref[...] = jnp.zeros_like(s2_ref)

    s1_ref[...] += jnp.sum(ep, axis=0, keepdims=True)
    s2_ref[...] += jnp.sum(ep * ep, axis=0, keepdims=True)


def _gather_body(h_hbm, src_hbm, xj_hbm, idx_v, rows_v, sem):
    c = lax.axis_index("c")
    s = lax.axis_index("s")
    wid = c * NS + s
    pltpu.sync_copy(src_hbm.at[wid], idx_v)

    def chunk(ci, carry):
        cps = [
            pltpu.async_copy(
                h_hbm.at[idx_v.at[ci * SPC + j]],
                rows_v.at[pl.ds(j * SUB, SUB)],
                sem,
            )
            for j in range(SPC)
        ]
        for cp in cps:
            cp.wait()
        pltpu.sync_copy(rows_v, xj_hbm.at[pl.ds(wid * EW + ci * CHUNK, CHUNK)])
        return carry

    lax.fori_loop(0, NCHUNK, chunk, 0)


def _msg_body(xj_ref, ep_ref, s1_ref, s2_ref, g_ref, b_ref, w2k_ref, rrep_ref,
              b2_ref, o_ref):
    inv_e = 1.0 / N_EDGES
    mu = s1_ref[...] * inv_e
    var = s2_ref[...] * inv_e - mu * mu
    ef = jnp.maximum((ep_ref[...] - mu) * lax.rsqrt(var + EPS) * g_ref[...]
                     + b_ref[...], 0.0)
    xj = xj_ref[...]
    u = jnp.dot(xj, w2k_ref[...], preferred_element_type=jnp.float32)
    efrep = jnp.dot(ef, rrep_ref[...], preferred_element_type=jnp.float32)
    p = efrep * u
    w = DIM * DIM
    while w > DIM:
        w //= 2
        p = p[:, :w] + p[:, w:]
    o_ref[...] = p + jnp.dot(xj, b2_ref[...], preferred_element_type=jnp.float32)


def _scatter_body(msg_hbm, dst_hbm, zeros_hbm, out_hbm, idx_v, buf, zbuf, acc,
                  sem):
    c = lax.axis_index("c")
    s = lax.axis_index("s")
    wid = c * NS + s
    pltpu.sync_copy(zeros_hbm, zbuf)
    pltpu.sync_copy(zbuf, acc.at[pl.ds(s * ROWS_PER_SUBCORE, ROWS_PER_SUBCORE)])
    plsc.subcore_barrier()

    pltpu.sync_copy(dst_hbm.at[wid], idx_v)

    def chunk(ci, carry):
        pltpu.sync_copy(msg_hbm.at[pl.ds(wid * EW + ci * CHUNK, CHUNK)], buf)

        def sub(j, carry2):
            pltpu.sync_copy(
                buf.at[pl.ds(j * SUB, SUB)],
                acc.at[idx_v.at[ci * SPC + j]],
                add=True,
            )
            return carry2

        lax.fori_loop(0, SPC, sub, 0)
        return carry

    lax.fori_loop(0, NCHUNK, chunk, 0)
    plsc.subcore_barrier()
    pltpu.sync_copy(
        acc.at[pl.ds(s * ROWS_PER_SUBCORE, ROWS_PER_SUBCORE)],
        out_hbm.at[c, pl.ds(s * ROWS_PER_SUBCORE, ROWS_PER_SUBCORE)],
    )


def _final_body(p_ref, g_ref, bb_ref, w_ref, b_ref, o_ref):
    agg = p_ref[0, :N_NODES, :] + p_ref[1, :N_NODES, :]
    mu = jnp.mean(agg, axis=0, keepdims=True)
    d = agg - mu
    var = jnp.mean(d * d, axis=0, keepdims=True)
    h2 = jnp.maximum(d * lax.rsqrt(var + EPS) * g_ref[...] + bb_ref[...], 0.0)
    s = jnp.sum(h2 * w_ref[...])
    o_ref[...] = jnp.full((1, 1), 1.0 / N_NODES) * s + b_ref[...]


def kernel(x, edge_index, edge_attr,
           lin0_W, lin0_b, bn0_g, bn0_b,
           nn1_W1, nn1_b1, nn1_bn_g, nn1_bn_b, nn1_W2, nn1_b2,
           bn1_g, bn1_b, lin1_W, lin1_b):
    f32 = jnp.float32
    src3 = edge_index[0].astype(jnp.int32).reshape(NW, NSUB, SUB)
    dst3 = edge_index[1].astype(jnp.int32).reshape(NW, NSUB, SUB)
    w2k = nn1_W2.reshape(DIM, DIM, DIM).transpose(1, 0, 2).reshape(DIM, DIM * DIM)
    rrep = jnp.repeat(jnp.eye(DIM, dtype=f32), DIM, axis=1)
    b2m = nn1_b2.reshape(DIM, DIM)

    h = pl.pallas_call(
        _lin0_body,
        out_shape=jax.ShapeDtypeStruct((N_NODES, DIM), f32),
    )(x, lin0_W, lin0_b.reshape(1, DIM), bn0_g.reshape(1, DIM),
      bn0_b.reshape(1, DIM))

    t2 = 2000
    g2 = N_EDGES // t2
    epre, s1, s2 = pl.pallas_call(
        _epre_body,
        grid=(g2,),
        in_specs=[
            pl.BlockSpec((t2, NUM_TYPES), lambda i: (i, 0)),
            pl.BlockSpec((NUM_TYPES, DIM), lambda i: (0, 0)),
            pl.BlockSpec((1, DIM), lambda i: (0, 0)),
        ],
        out_specs=[
            pl.BlockSpec((t2, DIM), lambda i: (i, 0)),
            pl.BlockSpec((1, DIM), lambda i: (0, 0)),
            pl.BlockSpec((1, DIM), lambda i: (0, 0)),
        ],
        out_shape=[
            jax.ShapeDtypeStruct((N_EDGES, DIM), f32),
            jax.ShapeDtypeStruct((1, DIM), f32),
            jax.ShapeDtypeStruct((1, DIM), f32),
        ],
    )(edge_attr, nn1_W1, nn1_b1.reshape(1, DIM))

    mesh = plsc.VectorSubcoreMesh(core_axis_name="c", subcore_axis_name="s",
                                  num_cores=NC, num_subcores=NS)
    sc_params = pltpu.CompilerParams(use_tc_tiling_on_sc=False)
    xj = pl.kernel(
        _gather_body,
        out_type=jax.ShapeDtypeStruct((N_EDGES, DIM), f32),
        mesh=mesh,
        compiler_params=sc_params,
        scratch_types=[
            pltpu.VMEM((NSUB, SUB), jnp.int32),
            pltpu.VMEM((CHUNK, DIM), f32),
            pltpu.SemaphoreType.DMA,
        ],
    )(h, src3)

    gm = N_EDGES // _MSG_T
    msg = pl.pallas_call(
        _msg_body,
        grid=(gm,),
        in_specs=[
            pl.BlockSpec((_MSG_T, DIM), lambda i: (i, 0)),
            pl.BlockSpec((_MSG_T, DIM), lambda i: (i, 0)),
            pl.BlockSpec((1, DIM), lambda i: (0, 0)),
            pl.BlockSpec((1, DIM), lambda i: (0, 0)),
            pl.BlockSpec((1, DIM), lambda i: (0, 0)),
            pl.BlockSpec((1, DIM), lambda i: (0, 0)),
            pl.BlockSpec((DIM, DIM * DIM), lambda i: (0, 0)),
            pl.BlockSpec((DIM, DIM * DIM), lambda i: (0, 0)),
            pl.BlockSpec((DIM, DIM), lambda i: (0, 0)),
        ],
        out_specs=pl.BlockSpec((_MSG_T, DIM), lambda i: (i, 0)),
        out_shape=jax.ShapeDtypeStruct((N_EDGES, DIM), f32),
    )(xj, epre, s1, s2, nn1_bn_g.reshape(1, DIM), nn1_bn_b.reshape(1, DIM),
      w2k, rrep, b2m)

    partials = pl.kernel(
        _scatter_body,
        out_type=jax.ShapeDtypeStruct((NC, N_PAD, DIM), f32),
        mesh=mesh,
        compiler_params=sc_params,
        scratch_types=[
            pltpu.VMEM((NSUB, SUB), jnp.int32),
            pltpu.VMEM((CHUNK, DIM), f32),
            pltpu.VMEM((ROWS_PER_SUBCORE, DIM), f32),
            pltpu.VMEM_SHARED((N_PAD, DIM), f32),
            pltpu.SemaphoreType.DMA,
        ],
    )(msg, dst3, jnp.zeros((ROWS_PER_SUBCORE, DIM), f32))

    out2d = pl.pallas_call(
        _final_body,
        out_shape=jax.ShapeDtypeStruct((1, 1), f32),
    )(partials, bn1_g.reshape(1, DIM), bn1_b.reshape(1, DIM),
      lin1_W.reshape(1, DIM), lin1_b.reshape(1, 1))
    return out2d[0, 0]


# speedup vs baseline: 3.2856x; 1.1223x over previous
import functools

import jax
import jax.numpy as jnp
from jax import lax
from jax.experimental import pallas as pl
from jax.experimental.pallas import tpu as pltpu
from jax.experimental.pallas import tpu_sc as plsc

N_NODES = 10000
N_EDGES = 160000
IN_DIM = 128
NUM_TYPES = 16
DIM = 32
EPS = 1e-5

NC, NS = 2, 16
NW = NC * NS
EW = N_EDGES // NW
SUB = 125
NSUB = EW // SUB
CHUNK = 1000
NCHUNK = EW // CHUNK
SPC = NSUB // NCHUNK
N_PAD = 10240
ROWS_PER_SUBCORE = N_PAD // NS

_MSG_T = 2000


def _lin0_body(x_ref, w_ref, b_ref, g_ref, bb_ref, o_ref):
    hp = jnp.dot(x_ref[...], w_ref[...], preferred_element_type=jnp.float32)
    hp = hp + b_ref[...]
    mu = jnp.mean(hp, axis=0, keepdims=True)
    d = hp - mu
    var = jnp.mean(d * d, axis=0, keepdims=True)
    o_ref[...] = jnp.maximum(d * lax.rsqrt(var + EPS) * g_ref[...] + bb_ref[...], 0.0)


def _epre_body(ea_ref, w_ref, b_ref, s1_ref, s2_ref):
    ep = jnp.dot(ea_ref[...], w_ref[...], preferred_element_type=jnp.float32)
    ep = ep + b_ref[...]

    @pl.when(pl.program_id(0) == 0)
    def _():
        s1_ref[...] = jnp.zeros_like(s1_ref)
        s2_ref[...] = jnp.zeros_like(s2_ref)

    s1_ref[...] += jnp.sum(ep, axis=0, keepdims=True)
    s2_ref[...] += jnp.sum(ep * ep, axis=0, keepdims=True)


def _gather_body(h_hbm, src_hbm, xj_hbm, idx_v, rows_v, sem):
    c = lax.axis_index("c")
    s = lax.axis_index("s")
    wid = c * NS + s
    pltpu.sync_copy(src_hbm.at[wid], idx_v)

    def chunk(ci, carry):
        cps = [
            pltpu.async_copy(
                h_hbm.at[idx_v.at[ci * SPC + j]],
                rows_v.at[pl.ds(j * SUB, SUB)],
                sem,
            )
            for j in range(SPC)
        ]
        for cp in cps:
            cp.wait()
        pltpu.sync_copy(rows_v, xj_hbm.at[pl.ds(wid * EW + ci * CHUNK, CHUNK)])
        return carry

    lax.fori_loop(0, NCHUNK, chunk, 0)


def _msg_body(xj_ref, ea_ref, w1_ref, b1_ref, s1_ref, s2_ref, g_ref, b_ref,
              w2k_ref, rrep_ref, b2_ref, o_ref):
    inv_e = 1.0 / N_EDGES
    mu = s1_ref[...] * inv_e
    var = s2_ref[...] * inv_e - mu * mu
    ep = jnp.dot(ea_ref[...], w1_ref[...], preferred_element_type=jnp.float32)
    ep = ep + b1_ref[...]
    ef = jnp.maximum((ep - mu) * lax.rsqrt(var + EPS) * g_ref[...]
                     + b_ref[...], 0.0)
    xj = xj_ref[...]
    u = jnp.dot(xj, w2k_ref[...], preferred_element_type=jnp.float32)
    efrep = jnp.dot(ef, rrep_ref[...], preferred_element_type=jnp.float32)
    p = efrep * u
    w = DIM * DIM
    while w > DIM:
        w //= 2
        p = p[:, :w] + p[:, w:]
    o_ref[...] = p + jnp.dot(xj, b2_ref[...], preferred_element_type=jnp.float32)


def _scatter_body(msg_hbm, dst_hbm, zeros_hbm, out_hbm, idx_v, buf, zbuf, acc,
                  sem):
    c = lax.axis_index("c")
    s = lax.axis_index("s")
    wid = c * NS + s
    pltpu.sync_copy(zeros_hbm, zbuf)
    pltpu.sync_copy(zbuf, acc.at[pl.ds(s * ROWS_PER_SUBCORE, ROWS_PER_SUBCORE)])
    plsc.subcore_barrier()

    pltpu.sync_copy(dst_hbm.at[wid], idx_v)

    def chunk(ci, carry):
        pltpu.sync_copy(msg_hbm.at[pl.ds(wid * EW + ci * CHUNK, CHUNK)], buf)

        def sub(j, carry2):
            pltpu.sync_copy(
                buf.at[pl.ds(j * SUB, SUB)],
                acc.at[idx_v.at[ci * SPC + j]],
                add=True,
            )
            return carry2

        lax.fori_loop(0, SPC, sub, 0)
        return carry

    lax.fori_loop(0, NCHUNK, chunk, 0)
    plsc.subcore_barrier()
    pltpu.sync_copy(
        acc.at[pl.ds(s * ROWS_PER_SUBCORE, ROWS_PER_SUBCORE)],
        out_hbm.at[c, pl.ds(s * ROWS_PER_SUBCORE, ROWS_PER_SUBCORE)],
    )


def _final_body(p_ref, g_ref, bb_ref, w_ref, b_ref, o_ref):
    agg = p_ref[0, :N_NODES, :] + p_ref[1, :N_NODES, :]
    mu = jnp.mean(agg, axis=0, keepdims=True)
    d = agg - mu
    var = jnp.mean(d * d, axis=0, keepdims=True)
    h2 = jnp.maximum(d * lax.rsqrt(var + EPS) * g_ref[...] + bb_ref[...], 0.0)
    s = jnp.sum(h2 * w_ref[...])
    o_ref[...] = jnp.full((1, 1), 1.0 / N_NODES) * s + b_ref[...]


def kernel(x, edge_index, edge_attr,
           lin0_W, lin0_b, bn0_g, bn0_b,
           nn1_W1, nn1_b1, nn1_bn_g, nn1_bn_b, nn1_W2, nn1_b2,
           bn1_g, bn1_b, lin1_W, lin1_b):
    f32 = jnp.float32
    src3 = edge_index[0].astype(jnp.int32).reshape(NW, NSUB, SUB)
    dst3 = edge_index[1].astype(jnp.int32).reshape(NW, NSUB, SUB)
    w2k = nn1_W2.reshape(DIM, DIM, DIM).transpose(1, 0, 2).reshape(DIM, DIM * DIM)
    rrep = jnp.repeat(jnp.eye(DIM, dtype=f32), DIM, axis=1)
    b2m = nn1_b2.reshape(DIM, DIM)

    h = pl.pallas_call(
        _lin0_body,
        out_shape=jax.ShapeDtypeStruct((N_NODES, DIM), f32),
    )(x, lin0_W, lin0_b.reshape(1, DIM), bn0_g.reshape(1, DIM),
      bn0_b.reshape(1, DIM))

    t2 = 4000
    g2 = N_EDGES // t2
    s1, s2 = pl.pallas_call(
        _epre_body,
        grid=(g2,),
        in_specs=[
            pl.BlockSpec((t2, NUM_TYPES), lambda i: (i, 0)),
            pl.BlockSpec((NUM_TYPES, DIM), lambda i: (0, 0)),
            pl.BlockSpec((1, DIM), lambda i: (0, 0)),
        ],
        out_specs=[
            pl.BlockSpec((1, DIM), lambda i: (0, 0)),
            pl.BlockSpec((1, DIM), lambda i: (0, 0)),
        ],
        out_shape=[
            jax.ShapeDtypeStruct((1, DIM), f32),
            jax.ShapeDtypeStruct((1, DIM), f32),
        ],
    )(edge_attr, nn1_W1, nn1_b1.reshape(1, DIM))

    mesh = plsc.VectorSubcoreMesh(core_axis_name="c", subcore_axis_name="s",
                                  num_cores=NC, num_subcores=NS)
    sc_params = pltpu.CompilerParams(use_tc_tiling_on_sc=False)
    xj = pl.kernel(
        _gather_body,
        out_type=jax.ShapeDtypeStruct((N_EDGES, DIM), f32),
        mesh=mesh,
        compiler_params=sc_params,
        scratch_types=[
            pltpu.VMEM((NSUB, SUB), jnp.int32),
            pltpu.VMEM((CHUNK, DIM), f32),
            pltpu.SemaphoreType.DMA,
        ],
    )(h, src3)

    gm = N_EDGES // _MSG_T
    msg = pl.pallas_call(
        _msg_body,
        grid=(gm,),
        in_specs=[
            pl.BlockSpec((_MSG_T, DIM), lambda i: (i, 0)),
            pl.BlockSpec((_MSG_T, NUM_TYPES), lambda i: (i, 0)),
            pl.BlockSpec((NUM_TYPES, DIM), lambda i: (0, 0)),
            pl.BlockSpec((1, DIM), lambda i: (0, 0)),
            pl.BlockSpec((1, DIM), lambda i: (0, 0)),
            pl.BlockSpec((1, DIM), lambda i: (0, 0)),
            pl.BlockSpec((1, DIM), lambda i: (0, 0)),
            pl.BlockSpec((1, DIM), lambda i: (0, 0)),
            pl.BlockSpec((DIM, DIM * DIM), lambda i: (0, 0)),
            pl.BlockSpec((DIM, DIM * DIM), lambda i: (0, 0)),
            pl.BlockSpec((DIM, DIM), lambda i: (0, 0)),
        ],
        out_specs=pl.BlockSpec((_MSG_T, DIM), lambda i: (i, 0)),
        out_shape=jax.ShapeDtypeStruct((N_EDGES, DIM), f32),
    )(xj, edge_attr, nn1_W1, nn1_b1.reshape(1, DIM), s1, s2,
      nn1_bn_g.reshape(1, DIM), nn1_bn_b.reshape(1, DIM), w2k, rrep, b2m)

    partials = pl.kernel(
        _scatter_body,
        out_type=jax.ShapeDtypeStruct((NC, N_PAD, DIM), f32),
        mesh=mesh,
        compiler_params=sc_params,
        scratch_types=[
            pltpu.VMEM((NSUB, SUB), jnp.int32),
            pltpu.VMEM((CHUNK, DIM), f32),
            pltpu.VMEM((ROWS_PER_SUBCORE, DIM), f32),
            pltpu.VMEM_SHARED((N_PAD, DIM), f32),
            pltpu.SemaphoreType.DMA,
        ],
    )(msg, dst3, jnp.zeros((ROWS_PER_SUBCORE, DIM), f32))

    out2d = pl.pallas_call(
        _final_body,
        out_shape=jax.ShapeDtypeStruct((1, 1), f32),
    )(partials, bn1_g.reshape(1, DIM), bn1_b.reshape(1, DIM),
      lin1_W.reshape(1, DIM), lin1_b.reshape(1, 1))
    return out2d[0, 0]


# speedup vs baseline: 4.0171x; 1.2226x over previous
import jax
import jax.numpy as jnp
from jax import lax
from jax.experimental import pallas as pl
from jax.experimental.pallas import tpu as pltpu
from jax.experimental.pallas import tpu_sc as plsc

N_NODES = 10000
N_EDGES = 160000
IN_DIM = 128
NUM_TYPES = 16
DIM = 32
EPS = 1e-5

PK = 8
EROWS = N_EDGES // PK

NC, NS = 2, 16
NW = NC * NS
EW = N_EDGES // NW
SUB = 125
NSUB = EW // SUB
CHUNK = 1000
NCHUNK = EW // CHUNK
SPC = NSUB // NCHUNK
N_PAD = 10240
ROWS_PER_SUBCORE = N_PAD // NS

_MSG_R = 200
_ST_R = 2000


def _lin0_body(x_ref, w_ref, b_ref, g_ref, bb_ref, o_ref):
    hp = jnp.dot(x_ref[...], w_ref[...], preferred_element_type=jnp.float32)
    hp = hp + b_ref[...]
    mu = jnp.mean(hp, axis=0, keepdims=True)
    d = hp - mu
    var = jnp.mean(d * d, axis=0, keepdims=True)
    o_ref[...] = jnp.maximum(d * lax.rsqrt(var + EPS) * g_ref[...] + bb_ref[...], 0.0)


def _stats_body(ea_ref, w1_ref, b1_ref, g_ref, bb_ref, a_ref, b_ref,
                macc, sacc):
    i = pl.program_id(0)
    blk = ea_ref[...]

    @pl.when(i == 0)
    def _():
        macc[...] = jnp.zeros_like(macc)
        sacc[...] = jnp.zeros_like(sacc)

    macc[...] += lax.dot_general(blk, blk, (((0,), (0,)), ((), ())),
                                 preferred_element_type=jnp.float32)
    sacc[...] += jnp.sum(blk, axis=0, keepdims=True)

    @pl.when(i == pl.num_programs(0) - 1)
    def _():
        m = macc[...]
        s = sacc[...]
        m16 = sum(m[NUM_TYPES * j:NUM_TYPES * (j + 1),
                    NUM_TYPES * j:NUM_TYPES * (j + 1)] for j in range(PK))
        s16 = sum(s[:, NUM_TYPES * j:NUM_TYPES * (j + 1)] for j in range(PK))
        inv_e = 1.0 / N_EDGES
        mu_x = jnp.dot(s16, w1_ref[...], preferred_element_type=jnp.float32) * inv_e
        ex2 = jnp.sum(w1_ref[...] * jnp.dot(m16, w1_ref[...],
                                            preferred_element_type=jnp.float32),
                      axis=0, keepdims=True) * inv_e
        var = ex2 - mu_x * mu_x
        mu = mu_x + b1_ref[...]
        a = lax.rsqrt(var + EPS) * g_ref[...]
        bc = bb_ref[...] - mu * a
        a_ref[...] = jnp.concatenate([a] * PK, axis=1)
        b_ref[...] = jnp.concatenate([bc] * PK, axis=1)


def _gather_body(h_hbm, src_hbm, xj_hbm, idx_v, rows_v, sem):
    c = lax.axis_index("c")
    s = lax.axis_index("s")
    wid = c * NS + s
    pltpu.sync_copy(src_hbm.at[wid], idx_v)

    def chunk(ci, carry):
        cps = [
            pltpu.async_copy(
                h_hbm.at[idx_v.at[ci * SPC + j]],
                rows_v.at[pl.ds(j * SUB, SUB)],
                sem,
            )
            for j in range(SPC)
        ]
        for cp in cps:
            cp.wait()
        pltpu.sync_copy(rows_v, xj_hbm.at[pl.ds(wid * EW + ci * CHUNK, CHUNK)])
        return carry

    lax.fori_loop(0, NCHUNK, chunk, 0)


def _msg_body(xj_ref, ea_ref, a_ref, b_ref, w2k8_ref, rep8_ref, epw8_ref,
              b28_ref, o_ref):
    ea8 = ea_ref[...]
    ep8 = jnp.dot(ea8, epw8_ref[...], preferred_element_type=jnp.float32)
    ef8 = jnp.maximum(ep8 * a_ref[...] + b_ref[...], 0.0)
    xj8 = xj_ref[...]
    u8 = jnp.dot(xj8.astype(jnp.bfloat16), w2k8_ref[...],
                 preferred_element_type=jnp.float32)
    efrep = jnp.dot(ef8.astype(jnp.bfloat16), rep8_ref[...],
                    preferred_element_type=jnp.float32)
    p = efrep * u8
    parts = []
    kk = DIM * DIM
    for j in range(PK):
        q = p[:, j * kk:(j + 1) * kk]
        w = kk
        while w > DIM:
            w //= 2
            q = q[:, :w] + q[:, w:]
        parts.append(q)
    msg8 = jnp.concatenate(parts, axis=1)
    o_ref[...] = msg8 + jnp.dot(xj8, b28_ref[...],
                                preferred_element_type=jnp.float32)


def _scatter_body(msg_hbm, dst_hbm, zeros_hbm, out_hbm, idx_v, buf, zbuf, acc,
                  sem):
    c = lax.axis_index("c")
    s = lax.axis_index("s")
    wid = c * NS + s
    pltpu.sync_copy(zeros_hbm, zbuf)
    pltpu.sync_copy(zbuf, acc.at[pl.ds(s * ROWS_PER_SUBCORE, ROWS_PER_SUBCORE)])
    plsc.subcore_barrier()

    pltpu.sync_copy(dst_hbm.at[wid], idx_v)

    def chunk(ci, carry):
        pltpu.sync_copy(msg_hbm.at[pl.ds(wid * EW + ci * CHUNK, CHUNK)], buf)

        def sub(j, carry2):
            pltpu.sync_copy(
                buf.at[pl.ds(j * SUB, SUB)],
                acc.at[idx_v.at[ci * SPC + j]],
                add=True,
            )
            return carry2

        lax.fori_loop(0, SPC, sub, 0)
        return carry

    lax.fori_loop(0, NCHUNK, chunk, 0)
    plsc.subcore_barrier()
    pltpu.sync_copy(
        acc.at[pl.ds(s * ROWS_PER_SUBCORE, ROWS_PER_SUBCORE)],
        out_hbm.at[c, pl.ds(s * ROWS_PER_SUBCORE, ROWS_PER_SUBCORE)],
    )


def _final_body(p_ref, g_ref, bb_ref, w_ref, b_ref, o_ref):
    nrows = N_NODES // PK
    agg = p_ref[0, :nrows, :] + p_ref[1, :nrows, :]
    sm = jnp.sum(agg, axis=0, keepdims=True)
    m32 = sum(sm[:, DIM * j:DIM * (j + 1)] for j in range(PK)) * (1.0 / N_NODES)
    mu = jnp.concatenate([m32] * PK, axis=1)
    d = agg - mu
    v = jnp.sum(d * d, axis=0, keepdims=True)
    v32 = sum(v[:, DIM * j:DIM * (j + 1)] for j in range(PK)) * (1.0 / N_NODES)
    var = jnp.concatenate([v32] * PK, axis=1)
    h2 = jnp.maximum(d * lax.rsqrt(var + EPS) * g_ref[...] + bb_ref[...], 0.0)
    s = jnp.sum(h2 * w_ref[...])
    o_ref[...] = jnp.full((1, 1), 1.0 / N_NODES) * s + b_ref[...]


def kernel(x, edge_index, edge_attr,
           lin0_W, lin0_b, bn0_g, bn0_b,
           nn1_W1, nn1_b1, nn1_bn_g, nn1_bn_b, nn1_W2, nn1_b2,
           bn1_g, bn1_b, lin1_W, lin1_b):
    f32 = jnp.float32
    bf16 = jnp.bfloat16
    src3 = edge_index[0].astype(jnp.int32).reshape(NW, NSUB, SUB)
    dst3 = edge_index[1].astype(jnp.int32).reshape(NW, NSUB, SUB)
    w2k = nn1_W2.reshape(DIM, DIM, DIM).transpose(1, 0, 2).reshape(DIM, DIM * DIM)
    eye8 = jnp.eye(PK, dtype=f32)
    w2k8 = jnp.kron(eye8, w2k).astype(bf16)
    rrep = jnp.repeat(jnp.eye(DIM, dtype=f32), DIM, axis=1)
    rep8 = jnp.kron(eye8, rrep).astype(bf16)
    epw8 = jnp.kron(eye8, nn1_W1)
    b28 = jnp.kron(eye8, nn1_b2.reshape(DIM, DIM))
    ea8 = edge_attr.reshape(EROWS, PK * NUM_TYPES)

    h = pl.pallas_call(
        _lin0_body,
        out_shape=jax.ShapeDtypeStruct((N_NODES, DIM), f32),
    )(x, lin0_W, lin0_b.reshape(1, DIM), bn0_g.reshape(1, DIM),
      bn0_b.reshape(1, DIM))

    g2 = EROWS // _ST_R
    a256, b256 = pl.pallas_call(
        _stats_body,
        grid=(g2,),
        in_specs=[
            pl.BlockSpec((_ST_R, PK * NUM_TYPES), lambda i: (i, 0)),
            pl.BlockSpec((NUM_TYPES, DIM), lambda i: (0, 0)),
            pl.BlockSpec((1, DIM), lambda i: (0, 0)),
            pl.BlockSpec((1, DIM), lambda i: (0, 0)),
            pl.BlockSpec((1, DIM), lambda i: (0, 0)),
        ],
        out_specs=[
            pl.BlockSpec((1, PK * DIM), lambda i: (0, 0)),
            pl.BlockSpec((1, PK * DIM), lambda i: (0, 0)),
        ],
        out_shape=[
            jax.ShapeDtypeStruct((1, PK * DIM), f32),
            jax.ShapeDtypeStruct((1, PK * DIM), f32),
        ],
        scratch_shapes=[
            pltpu.VMEM((PK * NUM_TYPES, PK * NUM_TYPES), f32),
            pltpu.VMEM((1, PK * NUM_TYPES), f32),
        ],
    )(ea8, nn1_W1, nn1_b1.reshape(1, DIM), nn1_bn_g.reshape(1, DIM),
      nn1_bn_b.reshape(1, DIM))

    mesh = plsc.VectorSubcoreMesh(core_axis_name="c", subcore_axis_name="s",
                                  num_cores=NC, num_subcores=NS)
    sc_params = pltpu.CompilerParams(use_tc_tiling_on_sc=False)
    xj = pl.kernel(
        _gather_body,
        out_type=jax.ShapeDtypeStruct((N_EDGES, DIM), f32),
        mesh=mesh,
        compiler_params=sc_params,
        scratch_types=[
            pltpu.VMEM((NSUB, SUB), jnp.int32),
            pltpu.VMEM((CHUNK, DIM), f32),
            pltpu.SemaphoreType.DMA,
        ],
    )(h, src3)
    xj8 = xj.reshape(EROWS, PK * DIM)

    gm = EROWS // _MSG_R
    msg8 = pl.pallas_call(
        _msg_body,
        grid=(gm,),
        in_specs=[
            pl.BlockSpec((_MSG_R, PK * DIM), lambda i: (i, 0)),
            pl.BlockSpec((_MSG_R, PK * NUM_TYPES), lambda i: (i, 0)),
            pl.BlockSpec((1, PK * DIM), lambda i: (0, 0)),
            pl.BlockSpec((1, PK * DIM), lambda i: (0, 0)),
            pl.BlockSpec((PK * DIM, PK * DIM * DIM), lambda i: (0, 0)),
            pl.BlockSpec((PK * DIM, PK * DIM * DIM), lambda i: (0, 0)),
            pl.BlockSpec((PK * NUM_TYPES, PK * DIM), lambda i: (0, 0)),
            pl.BlockSpec((PK * DIM, PK * DIM), lambda i: (0, 0)),
        ],
        out_specs=pl.BlockSpec((_MSG_R, PK * DIM), lambda i: (i, 0)),
        out_shape=jax.ShapeDtypeStruct((EROWS, PK * DIM), f32),
    )(xj8, ea8, a256, b256, w2k8, rep8, epw8, b28)
    msg = msg8.reshape(N_EDGES, DIM)

    partials = pl.kernel(
        _scatter_body,
        out_type=jax.ShapeDtypeStruct((NC, N_PAD, DIM), f32),
        mesh=mesh,
        compiler_params=sc_params,
        scratch_types=[
            pltpu.VMEM((NSUB, SUB), jnp.int32),
            pltpu.VMEM((CHUNK, DIM), f32),
            pltpu.VMEM((ROWS_PER_SUBCORE, DIM), f32),
            pltpu.VMEM_SHARED((N_PAD, DIM), f32),
            pltpu.SemaphoreType.DMA,
        ],
    )(msg, dst3, jnp.zeros((ROWS_PER_SUBCORE, DIM), f32))
    p8 = partials.reshape(NC, N_PAD // PK, PK * DIM)

    tile8 = lambda v: jnp.tile(v.reshape(1, DIM), (1, PK))
    out2d = pl.pallas_call(
        _final_body,
        out_shape=jax.ShapeDtypeStruct((1, 1), f32),
    )(p8, tile8(bn1_g), tile8(bn1_b), tile8(lin1_W), lin1_b.reshape(1, 1))
    return out2d[0, 0]


# speedup vs baseline: 4.0227x; 1.0014x over previous
import jax
import jax.numpy as jnp
from jax import lax
from jax.experimental import pallas as pl
from jax.experimental.pallas import tpu as pltpu
from jax.experimental.pallas import tpu_sc as plsc

N_NODES = 10000
N_EDGES = 160000
IN_DIM = 128
NUM_TYPES = 16
DIM = 32
EPS = 1e-5

PK = 8
EROWS = N_EDGES // PK

NC, NS = 2, 16
NW = NC * NS
EW = N_EDGES // NW
SUB = 125
NSUB = EW // SUB
CHUNK = 1000
NCHUNK = EW // CHUNK
SPC = NSUB // NCHUNK
N_PAD = 10240
ROWS_PER_SUBCORE = N_PAD // NS

_MSG_R = 200
_ST_R = 2000


def _lin0_body(x_ref, w_ref, b_ref, g_ref, bb_ref, o_ref):
    hp = jnp.dot(x_ref[...], w_ref[...], preferred_element_type=jnp.float32)
    hp = hp + b_ref[...]
    mu = jnp.mean(hp, axis=0, keepdims=True)
    d = hp - mu
    var = jnp.mean(d * d, axis=0, keepdims=True)
    o_ref[...] = jnp.maximum(d * lax.rsqrt(var + EPS) * g_ref[...] + bb_ref[...], 0.0)


def _stats_body(ea_ref, w1_ref, b1_ref, g_ref, bb_ref, a_ref, b_ref,
                macc, sacc):
    i = pl.program_id(0)
    blk = ea_ref[...]

    @pl.when(i == 0)
    def _():
        macc[...] = jnp.zeros_like(macc)
        sacc[...] = jnp.zeros_like(sacc)

    macc[...] += lax.dot_general(blk, blk, (((0,), (0,)), ((), ())),
                                 preferred_element_type=jnp.float32)
    sacc[...] += jnp.sum(blk, axis=0, keepdims=True)

    @pl.when(i == pl.num_programs(0) - 1)
    def _():
        m = macc[...]
        s = sacc[...]
        m16 = sum(m[NUM_TYPES * j:NUM_TYPES * (j + 1),
                    NUM_TYPES * j:NUM_TYPES * (j + 1)] for j in range(PK))
        s16 = sum(s[:, NUM_TYPES * j:NUM_TYPES * (j + 1)] for j in range(PK))
        inv_e = 1.0 / N_EDGES
        mu_x = jnp.dot(s16, w1_ref[...], preferred_element_type=jnp.float32) * inv_e
        ex2 = jnp.sum(w1_ref[...] * jnp.dot(m16, w1_ref[...],
                                            preferred_element_type=jnp.float32),
                      axis=0, keepdims=True) * inv_e
        var = ex2 - mu_x * mu_x
        mu = mu_x + b1_ref[...]
        a = lax.rsqrt(var + EPS) * g_ref[...]
        bc = bb_ref[...] - mu * a
        a_ref[...] = jnp.concatenate([a] * PK, axis=1)
        b_ref[...] = jnp.concatenate([bc] * PK, axis=1)


def _gather_body(h_hbm, src_hbm, xj_hbm, idx_v, rows_v, sem):
    c = lax.axis_index("c")
    s = lax.axis_index("s")
    wid = c * NS + s
    pltpu.sync_copy(src_hbm.at[wid], idx_v)

    def chunk(ci, carry):
        cps = [
            pltpu.async_copy(
                h_hbm.at[idx_v.at[ci * SPC + j]],
                rows_v.at[pl.ds(j * SUB, SUB)],
                sem,
            )
            for j in range(SPC)
        ]
        for cp in cps:
            cp.wait()
        pltpu.sync_copy(rows_v, xj_hbm.at[pl.ds(wid * EW + ci * CHUNK, CHUNK)])
        return carry

    lax.fori_loop(0, NCHUNK, chunk, 0)


def _msg_body(xj_ref, ea_ref, a_ref, b_ref, w2k8_ref, rep8_ref, epw8_ref,
              b28_ref, o_ref):
    ea8 = ea_ref[...]
    ep8 = jnp.dot(ea8, epw8_ref[...], preferred_element_type=jnp.float32)
    ef8 = jnp.maximum(ep8 * a_ref[...] + b_ref[...], 0.0)
    xj8 = xj_ref[...]
    u8 = jnp.dot(xj8.astype(jnp.bfloat16), w2k8_ref[...],
                 preferred_element_type=jnp.float32).astype(jnp.bfloat16)
    efrep = jnp.dot(ef8.astype(jnp.bfloat16), rep8_ref[...],
                    preferred_element_type=jnp.float32).astype(jnp.bfloat16)
    p = efrep * u8
    parts = []
    kk = DIM * DIM
    for j in range(PK):
        q = p[:, j * kk:(j + 1) * kk]
        w = kk
        while w > DIM:
            w //= 2
            q = q[:, :w] + q[:, w:]
        parts.append(q)
    msg8 = jnp.concatenate(parts, axis=1).astype(jnp.float32)
    o_ref[...] = msg8 + jnp.dot(xj8, b28_ref[...],
                                preferred_element_type=jnp.float32)


def _scatter_body(msg_hbm, dst_hbm, zeros_hbm, out_hbm, idx_v, buf, zbuf, acc,
                  sem):
    c = lax.axis_index("c")
    s = lax.axis_index("s")
    wid = c * NS + s
    pltpu.sync_copy(zeros_hbm, zbuf)
    pltpu.sync_copy(zbuf, acc.at[pl.ds(s * ROWS_PER_SUBCORE, ROWS_PER_SUBCORE)])
    plsc.subcore_barrier()

    pltpu.sync_copy(dst_hbm.at[wid], idx_v)

    def chunk(ci, carry):
        pltpu.sync_copy(msg_hbm.at[pl.ds(wid * EW + ci * CHUNK, CHUNK)], buf)

        def sub(j, carry2):
            pltpu.sync_copy(
                buf.at[pl.ds(j * SUB, SUB)],
                acc.at[idx_v.at[ci * SPC + j]],
                add=True,
            )
            return carry2

        lax.fori_loop(0, SPC, sub, 0)
        return carry

    lax.fori_loop(0, NCHUNK, chunk, 0)
    plsc.subcore_barrier()
    pltpu.sync_copy(
        acc.at[pl.ds(s * ROWS_PER_SUBCORE, ROWS_PER_SUBCORE)],
        out_hbm.at[c, pl.ds(s * ROWS_PER_SUBCORE, ROWS_PER_SUBCORE)],
    )


def _final_body(p_ref, g_ref, bb_ref, w_ref, b_ref, o_ref):
    nrows = N_NODES // PK
    agg = p_ref[0, :nrows, :] + p_ref[1, :nrows, :]
    sm = jnp.sum(agg, axis=0, keepdims=True)
    m32 = sum(sm[:, DIM * j:DIM * (j + 1)] for j in range(PK)) * (1.0 / N_NODES)
    mu = jnp.concatenate([m32] * PK, axis=1)
    d = agg - mu
    v = jnp.sum(d * d, axis=0, keepdims=True)
    v32 = sum(v[:, DIM * j:DIM * (j + 1)] for j in range(PK)) * (1.0 / N_NODES)
    var = jnp.concatenate([v32] * PK, axis=1)
    h2 = jnp.maximum(d * lax.rsqrt(var + EPS) * g_ref[...] + bb_ref[...], 0.0)
    s = jnp.sum(h2 * w_ref[...])
    o_ref[...] = jnp.full((1, 1), 1.0 / N_NODES) * s + b_ref[...]


def kernel(x, edge_index, edge_attr,
           lin0_W, lin0_b, bn0_g, bn0_b,
           nn1_W1, nn1_b1, nn1_bn_g, nn1_bn_b, nn1_W2, nn1_b2,
           bn1_g, bn1_b, lin1_W, lin1_b):
    f32 = jnp.float32
    bf16 = jnp.bfloat16
    src3 = edge_index[0].astype(jnp.int32).reshape(NW, NSUB, SUB)
    dst3 = edge_index[1].astype(jnp.int32).reshape(NW, NSUB, SUB)
    w2k = nn1_W2.reshape(DIM, DIM, DIM).transpose(1, 0, 2).reshape(DIM, DIM * DIM)
    eye8 = jnp.eye(PK, dtype=f32)
    w2k8 = jnp.kron(eye8, w2k).astype(bf16)
    rrep = jnp.repeat(jnp.eye(DIM, dtype=f32), DIM, axis=1)
    rep8 = jnp.kron(eye8, rrep).astype(bf16)
    epw8 = jnp.kron(eye8, nn1_W1)
    b28 = jnp.kron(eye8, nn1_b2.reshape(DIM, DIM))
    ea8 = edge_attr.reshape(EROWS, PK * NUM_TYPES)

    h = pl.pallas_call(
        _lin0_body,
        out_shape=jax.ShapeDtypeStruct((N_NODES, DIM), f32),
    )(x, lin0_W, lin0_b.reshape(1, DIM), bn0_g.reshape(1, DIM),
      bn0_b.reshape(1, DIM))

    g2 = EROWS // _ST_R
    a256, b256 = pl.pallas_call(
        _stats_body,
        grid=(g2,),
        in_specs=[
            pl.BlockSpec((_ST_R, PK * NUM_TYPES), lambda i: (i, 0)),
            pl.BlockSpec((NUM_TYPES, DIM), lambda i: (0, 0)),
            pl.BlockSpec((1, DIM), lambda i: (0, 0)),
            pl.BlockSpec((1, DIM), lambda i: (0, 0)),
            pl.BlockSpec((1, DIM), lambda i: (0, 0)),
        ],
        out_specs=[
            pl.BlockSpec((1, PK * DIM), lambda i: (0, 0)),
            pl.BlockSpec((1, PK * DIM), lambda i: (0, 0)),
        ],
        out_shape=[
            jax.ShapeDtypeStruct((1, PK * DIM), f32),
            jax.ShapeDtypeStruct((1, PK * DIM), f32),
        ],
        scratch_shapes=[
            pltpu.VMEM((PK * NUM_TYPES, PK * NUM_TYPES), f32),
            pltpu.VMEM((1, PK * NUM_TYPES), f32),
        ],
    )(ea8, nn1_W1, nn1_b1.reshape(1, DIM), nn1_bn_g.reshape(1, DIM),
      nn1_bn_b.reshape(1, DIM))

    mesh = plsc.VectorSubcoreMesh(core_axis_name="c", subcore_axis_name="s",
                                  num_cores=NC, num_subcores=NS)
    sc_params = pltpu.CompilerParams(use_tc_tiling_on_sc=False)
    xj = pl.kernel(
        _gather_body,
        out_type=jax.ShapeDtypeStruct((N_EDGES, DIM), f32),
        mesh=mesh,
        compiler_params=sc_params,
        scratch_types=[
            pltpu.VMEM((NSUB, SUB), jnp.int32),
            pltpu.VMEM((CHUNK, DIM), f32),
            pltpu.SemaphoreType.DMA,
        ],
    )(h, src3)
    xj8 = xj.reshape(EROWS, PK * DIM)

    gm = EROWS // _MSG_R
    msg8 = pl.pallas_call(
        _msg_body,
        grid=(gm,),
        in_specs=[
            pl.BlockSpec((_MSG_R, PK * DIM), lambda i: (i, 0)),
            pl.BlockSpec((_MSG_R, PK * NUM_TYPES), lambda i: (i, 0)),
            pl.BlockSpec((1, PK * DIM), lambda i: (0, 0)),
            pl.BlockSpec((1, PK * DIM), lambda i: (0, 0)),
            pl.BlockSpec((PK * DIM, PK * DIM * DIM), lambda i: (0, 0)),
            pl.BlockSpec((PK * DIM, PK * DIM * DIM), lambda i: (0, 0)),
            pl.BlockSpec((PK * NUM_TYPES, PK * DIM), lambda i: (0, 0)),
            pl.BlockSpec((PK * DIM, PK * DIM), lambda i: (0, 0)),
        ],
        out_specs=pl.BlockSpec((_MSG_R, PK * DIM), lambda i: (i, 0)),
        out_shape=jax.ShapeDtypeStruct((EROWS, PK * DIM), f32),
    )(xj8, ea8, a256, b256, w2k8, rep8, epw8, b28)
    msg = msg8.reshape(N_EDGES, DIM)

    partials = pl.kernel(
        _scatter_body,
        out_type=jax.ShapeDtypeStruct((NC, N_PAD, DIM), f32),
        mesh=mesh,
        compiler_params=sc_params,
        scratch_types=[
            pltpu.VMEM((NSUB, SUB), jnp.int32),
            pltpu.VMEM((CHUNK, DIM), f32),
            pltpu.VMEM((ROWS_PER_SUBCORE, DIM), f32),
            pltpu.VMEM_SHARED((N_PAD, DIM), f32),
            pltpu.SemaphoreType.DMA,
        ],
    )(msg, dst3, jnp.zeros((ROWS_PER_SUBCORE, DIM), f32))
    p8 = partials.reshape(NC, N_PAD // PK, PK * DIM)

    tile8 = lambda v: jnp.tile(v.reshape(1, DIM), (1, PK))
    out2d = pl.pallas_call(
        _final_body,
        out_shape=jax.ShapeDtypeStruct((1, 1), f32),
    )(p8, tile8(bn1_g), tile8(bn1_b), tile8(lin1_W), lin1_b.reshape(1, 1))
    return out2d[0, 0]


# speedup vs baseline: 4.2941x; 1.0675x over previous
import jax
import jax.numpy as jnp
from jax import lax
from jax.experimental import pallas as pl
from jax.experimental.pallas import tpu as pltpu
from jax.experimental.pallas import tpu_sc as plsc

N_NODES = 10000
N_EDGES = 160000
IN_DIM = 128
NUM_TYPES = 16
DIM = 32
EPS = 1e-5

PK = 8
EROWS = N_EDGES // PK

NC, NS = 2, 16
NW = NC * NS
EW = N_EDGES // NW
SUB = 125
NSUB = EW // SUB
CHUNK = 1000
NCHUNK = EW // CHUNK
SPC = NSUB // NCHUNK
N_PAD = 10240
ROWS_PER_SUBCORE = N_PAD // NS

_MSG_R = 400
_ST_R = 2000


def _lin0_body(x_ref, w_ref, b_ref, g_ref, bb_ref, o_ref):
    hp = jnp.dot(x_ref[...], w_ref[...], preferred_element_type=jnp.float32)
    hp = hp + b_ref[...]
    mu = jnp.mean(hp, axis=0, keepdims=True)
    d = hp - mu
    var = jnp.mean(d * d, axis=0, keepdims=True)
    o_ref[...] = jnp.maximum(d * lax.rsqrt(var + EPS) * g_ref[...] + bb_ref[...], 0.0)


def _stats_body(ea_ref, w1_ref, b1_ref, g_ref, bb_ref, a_ref, b_ref,
                macc, sacc):
    i = pl.program_id(0)
    blk = ea_ref[...]

    @pl.when(i == 0)
    def _():
        macc[...] = jnp.zeros_like(macc)
        sacc[...] = jnp.zeros_like(sacc)

    macc[...] += lax.dot_general(blk, blk, (((0,), (0,)), ((), ())),
                                 preferred_element_type=jnp.float32)
    sacc[...] += jnp.sum(blk, axis=0, keepdims=True)

    @pl.when(i == pl.num_programs(0) - 1)
    def _():
        m = macc[...]
        s = sacc[...]
        m16 = sum(m[NUM_TYPES * j:NUM_TYPES * (j + 1),
                    NUM_TYPES * j:NUM_TYPES * (j + 1)] for j in range(PK))
        s16 = sum(s[:, NUM_TYPES * j:NUM_TYPES * (j + 1)] for j in range(PK))
        inv_e = 1.0 / N_EDGES
        mu_x = jnp.dot(s16, w1_ref[...], preferred_element_type=jnp.float32) * inv_e
        ex2 = jnp.sum(w1_ref[...] * jnp.dot(m16, w1_ref[...],
                                            preferred_element_type=jnp.float32),
                      axis=0, keepdims=True) * inv_e
        var = ex2 - mu_x * mu_x
        mu = mu_x + b1_ref[...]
        a = lax.rsqrt(var + EPS) * g_ref[...]
        bc = bb_ref[...] - mu * a
        a_ref[...] = jnp.concatenate([a] * PK, axis=1)
        b_ref[...] = jnp.concatenate([bc] * PK, axis=1)


def _gather_body(h_hbm, src_hbm, xj_hbm, idx_v, rows_v, sem):
    c = lax.axis_index("c")
    s = lax.axis_index("s")
    wid = c * NS + s
    pltpu.sync_copy(src_hbm.at[wid], idx_v)

    def chunk(ci, carry):
        cps = [
            pltpu.async_copy(
                h_hbm.at[idx_v.at[ci * SPC + j]],
                rows_v.at[pl.ds(j * SUB, SUB)],
                sem,
            )
            for j in range(SPC)
        ]
        for cp in cps:
            cp.wait()
        pltpu.sync_copy(rows_v, xj_hbm.at[pl.ds(wid * EW + ci * CHUNK, CHUNK)])
        return carry

    lax.fori_loop(0, NCHUNK, chunk, 0)


def _msg_body(xj_ref, ea_ref, a_ref, b_ref, w2k8_ref, rep8_ref, epw8_ref,
              b28_ref, o_ref):
    ea8 = ea_ref[...]
    ep8 = jnp.dot(ea8, epw8_ref[...], preferred_element_type=jnp.float32)
    ef8 = jnp.maximum(ep8 * a_ref[...] + b_ref[...], 0.0)
    xj8 = xj_ref[...]
    u8 = jnp.dot(xj8.astype(jnp.bfloat16), w2k8_ref[...],
                 preferred_element_type=jnp.float32)
    efrep = jnp.dot(ef8.astype(jnp.bfloat16), rep8_ref[...],
                    preferred_element_type=jnp.float32)
    p = efrep * u8
    parts = []
    kk = DIM * DIM
    for j in range(PK):
        q = p[:, j * kk:(j + 1) * kk]
        w = kk
        while w > DIM:
            w //= 2
            q = q[:, :w] + q[:, w:]
        parts.append(q)
    msg8 = jnp.concatenate(parts, axis=1)
    o_ref[...] = msg8 + jnp.dot(xj8, b28_ref[...],
                                preferred_element_type=jnp.float32)


def _scatter_body(msg_hbm, dst_hbm, zeros_hbm, out_hbm, idx_v, buf, zbuf, acc,
                  sem):
    c = lax.axis_index("c")
    s = lax.axis_index("s")
    wid = c * NS + s
    pltpu.sync_copy(zeros_hbm, zbuf)
    pltpu.sync_copy(zbuf, acc.at[pl.ds(s * ROWS_PER_SUBCORE, ROWS_PER_SUBCORE)])
    plsc.subcore_barrier()

    pltpu.sync_copy(dst_hbm.at[wid], idx_v)

    def chunk(ci, carry):
        pltpu.sync_copy(msg_hbm.at[pl.ds(wid * EW + ci * CHUNK, CHUNK)], buf)

        def sub(j, carry2):
            pltpu.sync_copy(
                buf.at[pl.ds(j * SUB, SUB)],
                acc.at[idx_v.at[ci * SPC + j]],
                add=True,
            )
            return carry2

        lax.fori_loop(0, SPC, sub, 0)
        return carry

    lax.fori_loop(0, NCHUNK, chunk, 0)
    plsc.subcore_barrier()
    pltpu.sync_copy(
        acc.at[pl.ds(s * ROWS_PER_SUBCORE, ROWS_PER_SUBCORE)],
        out_hbm.at[c, pl.ds(s * ROWS_PER_SUBCORE, ROWS_PER_SUBCORE)],
    )


def _final_body(p_ref, g_ref, bb_ref, w_ref, b_ref, o_ref):
    nrows = N_NODES // PK
    agg = p_ref[0, :nrows, :] + p_ref[1, :nrows, :]
    sm = jnp.sum(agg, axis=0, keepdims=True)
    m32 = sum(sm[:, DIM * j:DIM * (j + 1)] for j in range(PK)) * (1.0 / N_NODES)
    mu = jnp.concatenate([m32] * PK, axis=1)
    d = agg - mu
    v = jnp.sum(d * d, axis=0, keepdims=True)
    v32 = sum(v[:, DIM * j:DIM * (j + 1)] for j in range(PK)) * (1.0 / N_NODES)
    var = jnp.concatenate([v32] * PK, axis=1)
    h2 = jnp.maximum(d * lax.rsqrt(var + EPS) * g_ref[...] + bb_ref[...], 0.0)
    s = jnp.sum(h2 * w_ref[...])
    o_ref[...] = jnp.full((1, 1), 1.0 / N_NODES) * s + b_ref[...]


def kernel(x, edge_index, edge_attr,
           lin0_W, lin0_b, bn0_g, bn0_b,
           nn1_W1, nn1_b1, nn1_bn_g, nn1_bn_b, nn1_W2, nn1_b2,
           bn1_g, bn1_b, lin1_W, lin1_b):
    f32 = jnp.float32
    bf16 = jnp.bfloat16
    src3 = edge_index[0].astype(jnp.int32).reshape(NW, NSUB, SUB)
    dst3 = edge_index[1].astype(jnp.int32).reshape(NW, NSUB, SUB)
    w2k = nn1_W2.reshape(DIM, DIM, DIM).transpose(1, 0, 2).reshape(DIM, DIM * DIM)
    eye8 = jnp.eye(PK, dtype=f32)
    w2k8 = jnp.kron(eye8, w2k).astype(bf16)
    rrep = jnp.repeat(jnp.eye(DIM, dtype=f32), DIM, axis=1)
    rep8 = jnp.kron(eye8, rrep).astype(bf16)
    epw8 = jnp.kron(eye8, nn1_W1)
    b28 = jnp.kron(eye8, nn1_b2.reshape(DIM, DIM))
    ea8 = edge_attr.reshape(EROWS, PK * NUM_TYPES)

    h = pl.pallas_call(
        _lin0_body,
        out_shape=jax.ShapeDtypeStruct((N_NODES, DIM), f32),
    )(x, lin0_W, lin0_b.reshape(1, DIM), bn0_g.reshape(1, DIM),
      bn0_b.reshape(1, DIM))

    g2 = EROWS // _ST_R
    a256, b256 = pl.pallas_call(
        _stats_body,
        grid=(g2,),
        in_specs=[
            pl.BlockSpec((_ST_R, PK * NUM_TYPES), lambda i: (i, 0)),
            pl.BlockSpec((NUM_TYPES, DIM), lambda i: (0, 0)),
            pl.BlockSpec((1, DIM), lambda i: (0, 0)),
            pl.BlockSpec((1, DIM), lambda i: (0, 0)),
            pl.BlockSpec((1, DIM), lambda i: (0, 0)),
        ],
        out_specs=[
            pl.BlockSpec((1, PK * DIM), lambda i: (0, 0)),
            pl.BlockSpec((1, PK * DIM), lambda i: (0, 0)),
        ],
        out_shape=[
            jax.ShapeDtypeStruct((1, PK * DIM), f32),
            jax.ShapeDtypeStruct((1, PK * DIM), f32),
        ],
        scratch_shapes=[
            pltpu.VMEM((PK * NUM_TYPES, PK * NUM_TYPES), f32),
            pltpu.VMEM((1, PK * NUM_TYPES), f32),
        ],
    )(ea8, nn1_W1, nn1_b1.reshape(1, DIM), nn1_bn_g.reshape(1, DIM),
      nn1_bn_b.reshape(1, DIM))

    mesh = plsc.VectorSubcoreMesh(core_axis_name="c", subcore_axis_name="s",
                                  num_cores=NC, num_subcores=NS)
    sc_params = pltpu.CompilerParams(use_tc_tiling_on_sc=False)
    xj = pl.kernel(
        _gather_body,
        out_type=jax.ShapeDtypeStruct((N_EDGES, DIM), f32),
        mesh=mesh,
        compiler_params=sc_params,
        scratch_types=[
            pltpu.VMEM((NSUB, SUB), jnp.int32),
            pltpu.VMEM((CHUNK, DIM), f32),
            pltpu.SemaphoreType.DMA,
        ],
    )(h, src3)
    xj8 = xj.reshape(EROWS, PK * DIM)

    gm = EROWS // _MSG_R
    msg8 = pl.pallas_call(
        _msg_body,
        grid=(gm,),
        in_specs=[
            pl.BlockSpec((_MSG_R, PK * DIM), lambda i: (i, 0)),
            pl.BlockSpec((_MSG_R, PK * NUM_TYPES), lambda i: (i, 0)),
            pl.BlockSpec((1, PK * DIM), lambda i: (0, 0)),
            pl.BlockSpec((1, PK * DIM), lambda i: (0, 0)),
            pl.BlockSpec((PK * DIM, PK * DIM * DIM), lambda i: (0, 0)),
            pl.BlockSpec((PK * DIM, PK * DIM * DIM), lambda i: (0, 0)),
            pl.BlockSpec((PK * NUM_TYPES, PK * DIM), lambda i: (0, 0)),
            pl.BlockSpec((PK * DIM, PK * DIM), lambda i: (0, 0)),
        ],
        out_specs=pl.BlockSpec((_MSG_R, PK * DIM), lambda i: (i, 0)),
        out_shape=jax.ShapeDtypeStruct((EROWS, PK * DIM), f32),
    )(xj8, ea8, a256, b256, w2k8, rep8, epw8, b28)
    msg = msg8.reshape(N_EDGES, DIM)

    partials = pl.kernel(
        _scatter_body,
        out_type=jax.ShapeDtypeStruct((NC, N_PAD, DIM), f32),
        mesh=mesh,
        compiler_params=sc_params,
        scratch_types=[
            pltpu.VMEM((NSUB, SUB), jnp.int32),
            pltpu.VMEM((CHUNK, DIM), f32),
            pltpu.VMEM((ROWS_PER_SUBCORE, DIM), f32),
            pltpu.VMEM_SHARED((N_PAD, DIM), f32),
            pltpu.SemaphoreType.DMA,
        ],
    )(msg, dst3, jnp.zeros((ROWS_PER_SUBCORE, DIM), f32))
    p8 = partials.reshape(NC, N_PAD // PK, PK * DIM)

    tile8 = lambda v: jnp.tile(v.reshape(1, DIM), (1, PK))
    out2d = pl.pallas_call(
        _final_body,
        out_shape=jax.ShapeDtypeStruct((1, 1), f32),
    )(p8, tile8(bn1_g), tile8(bn1_b), tile8(lin1_W), lin1_b.reshape(1, 1))
    return out2d[0, 0]
